# baseline (speedup 1.0000x reference)
import functools

import jax
import jax.numpy as jnp
import numpy as np
from jax import lax
from jax.experimental import pallas as pl
from jax.experimental.pallas import tpu as pltpu
from jax.experimental.pallas import tpu_sc as plsc

D_MODEL = 1024
MAX_POS = 2048
BATCH = 4
SEQ = 2048
SCALE = 32.0

NC = 2
NS = 16
NW = NC * NS
S_PER_W = SEQ // NW
R = 32
LANES = 16


def _positional_encoding(max_pos, d_model):
    pos = np.arange(max_pos)[:, np.newaxis].astype(np.float32)
    i = np.arange(d_model)[np.newaxis, :].astype(np.float32)
    angle_rates = 1.0 / np.power(
        10000.0, 2.0 * (np.floor(i / 2.0)) / np.float32(d_model))
    angle_rads = pos * angle_rates
    angle_rads[:, 0::2] = np.sin(angle_rads[:, 0::2])
    angle_rads[:, 1::2] = np.cos(angle_rads[:, 1::2])
    return jnp.asarray(angle_rads, dtype=jnp.float32)


_PE_CONST = _positional_encoding(MAX_POS, D_MODEL)

_mesh = plsc.VectorSubcoreMesh(core_axis_name="c", subcore_axis_name="s")


@functools.partial(
    pl.kernel,
    mesh=_mesh,
    out_type=jax.ShapeDtypeStruct((BATCH * SEQ, D_MODEL), jnp.float32),
    scratch_types=[
        pltpu.VMEM((R,), jnp.int32),
        pltpu.VMEM((R, D_MODEL), jnp.float32),
        pltpu.VMEM((S_PER_W, D_MODEL), jnp.float32),
        pltpu.SemaphoreType.DMA,
    ],
)
def _emb_kernel(x_hbm, table_hbm, pe_hbm, out_hbm, idx_v, rows_v, pe_v, sem):
    wid = lax.axis_index("s") * NC + lax.axis_index("c")
    s0 = wid * S_PER_W
    pltpu.sync_copy(pe_hbm.at[pl.ds(s0, S_PER_W)], pe_v)

    def chunk_body(i, _):
        b = i // (S_PER_W // R)
        off = (i % (S_PER_W // R)) * R
        base = b * SEQ + s0 + off
        pltpu.sync_copy(x_hbm.at[pl.ds(base, R)], idx_v)
        pltpu.async_copy(table_hbm.at[idx_v], rows_v, sem).wait()

        def row_body(r, _):
            def col_body(c, _):
                sl = pl.ds(c * LANES, LANES)
                rows_v[r, sl] = rows_v[r, sl] * SCALE + pe_v[off + r, sl]
                return 0
            lax.fori_loop(0, D_MODEL // LANES, col_body, 0)
            return 0
        lax.fori_loop(0, R, row_body, 0)

        pltpu.sync_copy(rows_v, out_hbm.at[pl.ds(base, R)])
        return 0

    lax.fori_loop(0, BATCH * (S_PER_W // R), chunk_body, 0)


def kernel(x, training, table):
    xf = x.reshape(-1).astype(jnp.int32)
    out = _emb_kernel(xf, table, _PE_CONST)
    return out.reshape(BATCH, SEQ, D_MODEL)


# speedup vs baseline: 1.8626x; 1.8626x over previous
import functools

import jax
import jax.numpy as jnp
import numpy as np
from jax import lax
from jax.experimental import pallas as pl
from jax.experimental.pallas import tpu as pltpu
from jax.experimental.pallas import tpu_sc as plsc

D_MODEL = 1024
MAX_POS = 2048
BATCH = 4
SEQ = 2048
SCALE = 32.0

NC = 2
NS = 16
NW = NC * NS
S_PER_W = SEQ // NW
R = 16
NCHUNK = BATCH * S_PER_W // R
NBUF = 3
LANES = 16


def _positional_encoding(max_pos, d_model):
    pos = np.arange(max_pos)[:, np.newaxis].astype(np.float32)
    i = np.arange(d_model)[np.newaxis, :].astype(np.float32)
    angle_rates = 1.0 / np.power(
        10000.0, 2.0 * (np.floor(i / 2.0)) / np.float32(d_model))
    angle_rads = pos * angle_rates
    angle_rads[:, 0::2] = np.sin(angle_rads[:, 0::2])
    angle_rads[:, 1::2] = np.cos(angle_rads[:, 1::2])
    return jnp.asarray(angle_rads, dtype=jnp.float32)


_PE_CONST = _positional_encoding(MAX_POS, D_MODEL)

_mesh = plsc.VectorSubcoreMesh(core_axis_name="c", subcore_axis_name="s")


@functools.partial(
    pl.kernel,
    mesh=_mesh,
    out_type=jax.ShapeDtypeStruct((BATCH * SEQ, D_MODEL), jnp.float32),
    scratch_types=(
        [pltpu.VMEM((BATCH * S_PER_W,), jnp.int32)]
        + [pltpu.VMEM((R, D_MODEL), jnp.float32) for _ in range(NBUF)]
        + [pltpu.VMEM((S_PER_W, D_MODEL), jnp.float32)]
        + [pltpu.SemaphoreType.DMA for _ in range(2 * NBUF)]
    ),
)
def _emb_kernel(x_hbm, table_hbm, pe_hbm, out_hbm, idx_v,
                buf0, buf1, buf2, pe_v,
                g0, g1, g2, s0_, s1_, s2_):
    bufs = (buf0, buf1, buf2)
    gsems = (g0, g1, g2)
    ssems = (s0_, s1_, s2_)

    wid = lax.axis_index("s") * NC + lax.axis_index("c")
    s0 = wid * S_PER_W

    pltpu.sync_copy(pe_hbm.at[pl.ds(s0, S_PER_W)], pe_v)
    for b in range(BATCH):
        pltpu.sync_copy(x_hbm.at[pl.ds(b * SEQ + s0, S_PER_W)],
                        idx_v.at[pl.ds(b * S_PER_W, S_PER_W)])

    def fire_gather(c):
        return pltpu.async_copy(
            table_hbm.at[idx_v.at[pl.ds(c * R, R)]],
            bufs[c % NBUF], gsems[c % NBUF])

    ghandles = [None] * NCHUNK
    shandles = [None] * NCHUNK
    ghandles[0] = fire_gather(0)
    ghandles[1] = fire_gather(1)

    for c in range(NCHUNK):
        b, j = divmod(c, S_PER_W // R)
        base = b * SEQ + s0 + j * R
        buf = bufs[c % NBUF]

        ghandles[c].wait()

        def row_body(r, _):
            pe_r = j * R + r
            for col in range(D_MODEL // LANES):
                sl = pl.ds(col * LANES, LANES)
                buf[r, sl] = buf[r, sl] * SCALE + pe_v[pe_r, sl]
            return 0
        lax.fori_loop(0, R, row_body, 0, unroll=False)

        shandles[c] = pltpu.async_copy(
            buf, out_hbm.at[pl.ds(base, R)], ssems[c % NBUF])

        if c + 2 < NCHUNK:
            if c >= 1:
                shandles[c - 1].wait()
            ghandles[c + 2] = fire_gather(c + 2)

    for c in range(NCHUNK - 3, NCHUNK):
        shandles[c].wait()


def kernel(x, training, table):
    xf = x.reshape(-1).astype(jnp.int32)
    out = _emb_kernel(xf, table, _PE_CONST)
    return out.reshape(BATCH, SEQ, D_MODEL)
